# Initial kernel scaffold; baseline (speedup 1.0000x reference)
import functools

import jax
import jax.numpy as jnp
from jax import lax
from jax.experimental import pallas as pl
from jax.experimental.pallas import tpu as pltpu
from jax.experimental.pallas import tpu_sc as plsc

N_NODES = 10000
N_EDGES = 320000

NC = 2
NS = 16
NW = NC * NS
CH = 128

E_PAD = ((N_EDGES + NW * CH - 1) // (NW * CH)) * (NW * CH)
R_EDGE = E_PAD // CH
RPW = R_EDGE // NW

N_PAD = 10240
SLICE = N_PAD // NS
TRASH = N_NODES

_mesh = plsc.VectorSubcoreMesh(core_axis_name="c", subcore_axis_name="s")


def _sc_degree(dst2d, ones_hbm, zeros_hbm):
    @functools.partial(
        pl.kernel,
        out_type=jax.ShapeDtypeStruct((NC, N_PAD), jnp.float32),
        mesh=_mesh,
        scratch_types=[
            pltpu.VMEM((RPW, CH), jnp.int32),
            pltpu.VMEM((CH,), jnp.float32),
            pltpu.VMEM_SHARED((N_PAD,), jnp.float32),
        ],
    )
    def k(dst_hbm, ones_h, zeros_h, out_hbm, idx_v, ones_v, acc_s):
        c = lax.axis_index("c")
        s = lax.axis_index("s")
        w = s * NC + c
        pltpu.sync_copy(zeros_h.at[pl.ds(s * SLICE, SLICE)],
                        acc_s.at[pl.ds(s * SLICE, SLICE)])
        pltpu.sync_copy(ones_h, ones_v)
        pltpu.sync_copy(dst_hbm.at[pl.ds(w * RPW, RPW)], idx_v)
        plsc.subcore_barrier()

        @pl.loop(0, RPW)
        def _(j):
            pltpu.sync_copy(ones_v, acc_s.at[idx_v.at[j]], add=True)

        plsc.subcore_barrier()
        pltpu.sync_copy(acc_s.at[pl.ds(s * SLICE, SLICE)],
                        out_hbm.at[c, pl.ds(s * SLICE, SLICE)])

    return k(dst2d, ones_hbm, zeros_hbm)


def _sc_aggregate(g, src2d, dst2d, zeros_hbm, d):
    @functools.partial(
        pl.kernel,
        out_type=jax.ShapeDtypeStruct((NC, N_PAD, d), jnp.float32),
        mesh=_mesh,
        scratch_types=[
            pltpu.VMEM((RPW, CH), jnp.int32),
            pltpu.VMEM((RPW, CH), jnp.int32),
            pltpu.VMEM((CH, d), jnp.float32),
            pltpu.VMEM_SHARED((N_PAD, d), jnp.float32),
            pltpu.SemaphoreType.DMA,
        ],
    )
    def k(g_hbm, src_hbm, dst_hbm, zeros_h, out_hbm,
          isrc_v, idst_v, rows_v, acc_s, sem):
        c = lax.axis_index("c")
        s = lax.axis_index("s")
        w = s * NC + c
        pltpu.sync_copy(zeros_h, acc_s.at[pl.ds(s * SLICE, SLICE)])
        pltpu.sync_copy(src_hbm.at[pl.ds(w * RPW, RPW)], isrc_v)
        pltpu.sync_copy(dst_hbm.at[pl.ds(w * RPW, RPW)], idst_v)
        plsc.subcore_barrier()

        @pl.loop(0, RPW)
        def _(j):
            pltpu.async_copy(g_hbm.at[isrc_v.at[j]], rows_v, sem).wait()
            pltpu.sync_copy(rows_v, acc_s.at[idst_v.at[j]], add=True)

        plsc.subcore_barrier()
        pltpu.sync_copy(acc_s.at[pl.ds(s * SLICE, SLICE)],
                        out_hbm.at[c, pl.ds(s * SLICE, SLICE)])

    return k(g, src2d, dst2d, zeros_hbm)


_BLK = 2000


def _tc_layer1(x, W1, p0, p1):
    def body(x_ref, w_ref, p0_ref, p1_ref, g_ref, dinv_ref):
        deg = p0_ref[...] + p1_ref[...] + 1.0
        dinv = lax.rsqrt(deg)
        h = jnp.dot(x_ref[...], w_ref[...],
                    preferred_element_type=jnp.float32,
                    precision=lax.Precision.HIGHEST)
        g_ref[...] = h * dinv
        dinv_ref[...] = dinv

    n, f = x.shape
    fo = W1.shape[1]
    return pl.pallas_call(
        body,
        grid=(n // _BLK,),
        in_specs=[
            pl.BlockSpec((_BLK, f), lambda i: (i, 0)),
            pl.BlockSpec((f, fo), lambda i: (0, 0)),
            pl.BlockSpec((_BLK, 1), lambda i: (i, 0)),
            pl.BlockSpec((_BLK, 1), lambda i: (i, 0)),
        ],
        out_specs=[
            pl.BlockSpec((_BLK, fo), lambda i: (i, 0)),
            pl.BlockSpec((_BLK, 1), lambda i: (i, 0)),
        ],
        out_shape=[
            jax.ShapeDtypeStruct((n, fo), jnp.float32),
            jax.ShapeDtypeStruct((n, 1), jnp.float32),
        ],
    )(x, W1, p0, p1)


def _tc_layer2(a0, a1, g1, dinv, b1, W2):
    def body(a0_ref, a1_ref, g1_ref, dinv_ref, b1_ref, w_ref, out_ref):
        dinv = dinv_ref[...]
        pre = (a0_ref[...] + a1_ref[...] + g1_ref[...]) * dinv + b1_ref[...]
        h = jnp.maximum(pre, 0.0)
        out_ref[...] = jnp.dot(h, w_ref[...],
                               preferred_element_type=jnp.float32,
                               precision=lax.Precision.HIGHEST) * dinv

    n, f = g1.shape
    fo = W2.shape[1]
    return pl.pallas_call(
        body,
        grid=(n // _BLK,),
        in_specs=[
            pl.BlockSpec((_BLK, f), lambda i: (i, 0)),
            pl.BlockSpec((_BLK, f), lambda i: (i, 0)),
            pl.BlockSpec((_BLK, f), lambda i: (i, 0)),
            pl.BlockSpec((_BLK, 1), lambda i: (i, 0)),
            pl.BlockSpec((1, f), lambda i: (0, 0)),
            pl.BlockSpec((f, fo), lambda i: (0, 0)),
        ],
        out_specs=pl.BlockSpec((_BLK, fo), lambda i: (i, 0)),
        out_shape=jax.ShapeDtypeStruct((n, fo), jnp.float32),
    )(a0, a1, g1, dinv, b1, W2)


def _tc_final(a0, a1, g2, dinv, b2):
    def body(a0_ref, a1_ref, g2_ref, dinv_ref, b2_ref, out_ref):
        out_ref[...] = ((a0_ref[...] + a1_ref[...] + g2_ref[...])
                        * dinv_ref[...] + b2_ref[...])

    n, f = g2.shape
    return pl.pallas_call(
        body,
        grid=(n // _BLK,),
        in_specs=[
            pl.BlockSpec((_BLK, f), lambda i: (i, 0)),
            pl.BlockSpec((_BLK, f), lambda i: (i, 0)),
            pl.BlockSpec((_BLK, f), lambda i: (i, 0)),
            pl.BlockSpec((_BLK, 1), lambda i: (i, 0)),
            pl.BlockSpec((1, f), lambda i: (0, 0)),
        ],
        out_specs=pl.BlockSpec((_BLK, f), lambda i: (i, 0)),
        out_shape=jax.ShapeDtypeStruct((n, f), jnp.float32),
    )(a0, a1, g2, dinv, b2)


def kernel(x, edge_index, W1, b1, W2, b2):
    n = x.shape[0]
    f1 = W1.shape[1]
    f2 = W2.shape[1]

    src = edge_index[0].astype(jnp.int32)
    dst = edge_index[1].astype(jnp.int32)
    pad = E_PAD - src.shape[0]
    src2d = jnp.concatenate([src, jnp.zeros((pad,), jnp.int32)]).reshape(R_EDGE, CH)
    dst2d = jnp.concatenate([dst, jnp.full((pad,), TRASH, jnp.int32)]).reshape(R_EDGE, CH)

    ones_hbm = jnp.ones((CH,), jnp.float32)
    zdeg = jnp.zeros((N_PAD,), jnp.float32)
    z1 = jnp.zeros((SLICE, f1), jnp.float32)
    z2 = jnp.zeros((SLICE, f2), jnp.float32)

    deg_parts = _sc_degree(dst2d, ones_hbm, zdeg)
    p0 = deg_parts[0, :n, None]
    p1 = deg_parts[1, :n, None]

    g1, dinv = _tc_layer1(x, W1, p0, p1)

    acc1 = _sc_aggregate(g1, src2d, dst2d, z1, f1)
    g2 = _tc_layer2(acc1[0, :n], acc1[1, :n], g1, dinv,
                    b1.reshape(1, f1), W2)

    acc2 = _sc_aggregate(g2, src2d, dst2d, z2, f2)
    out = _tc_final(acc2[0, :n], acc2[1, :n], g2, dinv, b2.reshape(1, f2))
    return out


# speedup vs baseline: 21.3432x; 21.3432x over previous
import functools

import jax
import jax.numpy as jnp
from jax import lax
from jax.experimental import pallas as pl
from jax.experimental.pallas import tpu as pltpu
from jax.experimental.pallas import tpu_sc as plsc

N_NODES = 10000
N_EDGES = 320000

NC = 2
NS = 16
NW = NC * NS
CH = 128

RPW = ((N_EDGES + NW * CH - 1) // (NW * CH) + 7) // 8 * 8
R_EDGE = RPW * NW
E_PAD = R_EDGE * CH

N_PAD = 10240
SLICE = N_PAD // NS
TRASH = N_NODES

_mesh = plsc.VectorSubcoreMesh(core_axis_name="c", subcore_axis_name="s",
                               num_cores=NC, num_subcores=NS)


def _sc_degree(dst2d, ones_hbm, zeros_hbm):
    @functools.partial(
        pl.kernel,
        out_type=jax.ShapeDtypeStruct((NC * N_PAD,), jnp.float32),
        mesh=_mesh,
        scratch_types=[
            pltpu.VMEM((RPW, CH), jnp.int32),
            pltpu.VMEM((CH,), jnp.float32),
            pltpu.VMEM_SHARED((N_PAD,), jnp.float32),
        ],
    )
    def k(dst_hbm, ones_h, zeros_h, out_hbm, idx_v, ones_v, acc_s):
        c = lax.axis_index("c")
        s = lax.axis_index("s")
        w = s * NC + c
        pltpu.sync_copy(zeros_h.at[pl.ds(s * SLICE, SLICE)],
                        acc_s.at[pl.ds(s * SLICE, SLICE)])
        pltpu.sync_copy(ones_h, ones_v)
        pltpu.sync_copy(dst_hbm.at[pl.ds(w * RPW, RPW)], idx_v)
        plsc.subcore_barrier()

        @pl.loop(0, RPW)
        def _(j):
            pltpu.sync_copy(ones_v, acc_s.at[idx_v.at[j]], add=True)

        plsc.subcore_barrier()
        pltpu.sync_copy(acc_s.at[pl.ds(s * SLICE, SLICE)],
                        out_hbm.at[pl.ds(c * N_PAD + s * SLICE, SLICE)])

    return k(dst2d, ones_hbm, zeros_hbm)


def _sc_aggregate(g, src2d, dst2d, zeros_hbm, d):
    @functools.partial(
        pl.kernel,
        out_type=jax.ShapeDtypeStruct((NC, N_PAD, d), jnp.float32),
        mesh=_mesh,
        scratch_types=[
            pltpu.VMEM((RPW, CH), jnp.int32),
            pltpu.VMEM((RPW, CH), jnp.int32),
            pltpu.VMEM((CH, d), jnp.float32),
            pltpu.VMEM_SHARED((N_PAD, d), jnp.float32),
            pltpu.SemaphoreType.DMA,
        ],
    )
    def k(g_hbm, src_hbm, dst_hbm, zeros_h, out_hbm,
          isrc_v, idst_v, rows_v, acc_s, sem):
        c = lax.axis_index("c")
        s = lax.axis_index("s")
        w = s * NC + c
        pltpu.sync_copy(zeros_h, acc_s.at[pl.ds(s * SLICE, SLICE)])
        pltpu.sync_copy(src_hbm.at[pl.ds(w * RPW, RPW)], isrc_v)
        pltpu.sync_copy(dst_hbm.at[pl.ds(w * RPW, RPW)], idst_v)
        plsc.subcore_barrier()

        @pl.loop(0, RPW)
        def _(j):
            pltpu.async_copy(g_hbm.at[isrc_v.at[j]], rows_v, sem).wait()
            pltpu.sync_copy(rows_v, acc_s.at[idst_v.at[j]], add=True)

        plsc.subcore_barrier()
        pltpu.sync_copy(acc_s.at[pl.ds(s * SLICE, SLICE)],
                        out_hbm.at[c, pl.ds(s * SLICE, SLICE)])

    return k(g, src2d, dst2d, zeros_hbm)


_BLK = 2000


def _tc_layer1(x, W1, p0, p1):
    def body(x_ref, w_ref, p0_ref, p1_ref, g_ref, dinv_ref):
        deg = p0_ref[...] + p1_ref[...] + 1.0
        dinv = lax.rsqrt(deg)
        h = jnp.dot(x_ref[...], w_ref[...],
                    preferred_element_type=jnp.float32,
                    precision=lax.Precision.HIGHEST)
        g_ref[...] = h * dinv
        dinv_ref[...] = dinv

    n, f = x.shape
    fo = W1.shape[1]
    return pl.pallas_call(
        body,
        grid=(n // _BLK,),
        in_specs=[
            pl.BlockSpec((_BLK, f), lambda i: (i, 0)),
            pl.BlockSpec((f, fo), lambda i: (0, 0)),
            pl.BlockSpec((_BLK, 1), lambda i: (i, 0)),
            pl.BlockSpec((_BLK, 1), lambda i: (i, 0)),
        ],
        out_specs=[
            pl.BlockSpec((_BLK, fo), lambda i: (i, 0)),
            pl.BlockSpec((_BLK, 1), lambda i: (i, 0)),
        ],
        out_shape=[
            jax.ShapeDtypeStruct((n, fo), jnp.float32),
            jax.ShapeDtypeStruct((n, 1), jnp.float32),
        ],
    )(x, W1, p0, p1)


def _tc_layer2(a0, a1, g1, dinv, b1, W2):
    def body(a0_ref, a1_ref, g1_ref, dinv_ref, b1_ref, w_ref, out_ref):
        dinv = dinv_ref[...]
        pre = (a0_ref[...] + a1_ref[...] + g1_ref[...]) * dinv + b1_ref[...]
        h = jnp.maximum(pre, 0.0)
        out_ref[...] = jnp.dot(h, w_ref[...],
                               preferred_element_type=jnp.float32,
                               precision=lax.Precision.HIGHEST) * dinv

    n, f = g1.shape
    fo = W2.shape[1]
    return pl.pallas_call(
        body,
        grid=(n // _BLK,),
        in_specs=[
            pl.BlockSpec((_BLK, f), lambda i: (i, 0)),
            pl.BlockSpec((_BLK, f), lambda i: (i, 0)),
            pl.BlockSpec((_BLK, f), lambda i: (i, 0)),
            pl.BlockSpec((_BLK, 1), lambda i: (i, 0)),
            pl.BlockSpec((1, f), lambda i: (0, 0)),
            pl.BlockSpec((f, fo), lambda i: (0, 0)),
        ],
        out_specs=pl.BlockSpec((_BLK, fo), lambda i: (i, 0)),
        out_shape=jax.ShapeDtypeStruct((n, fo), jnp.float32),
    )(a0, a1, g1, dinv, b1, W2)


def _tc_final(a0, a1, g2, dinv, b2, fo):
    def body(a0_ref, a1_ref, g2_ref, dinv_ref, b2_ref, out_ref):
        s = (a0_ref[...] + a1_ref[...] + g2_ref[...])[:, :fo]
        out_ref[...] = s * dinv_ref[...] + b2_ref[...]

    n, f = g2.shape
    return pl.pallas_call(
        body,
        grid=(n // _BLK,),
        in_specs=[
            pl.BlockSpec((_BLK, f), lambda i: (i, 0)),
            pl.BlockSpec((_BLK, f), lambda i: (i, 0)),
            pl.BlockSpec((_BLK, f), lambda i: (i, 0)),
            pl.BlockSpec((_BLK, 1), lambda i: (i, 0)),
            pl.BlockSpec((1, fo), lambda i: (0, 0)),
        ],
        out_specs=pl.BlockSpec((_BLK, fo), lambda i: (i, 0)),
        out_shape=jax.ShapeDtypeStruct((n, fo), jnp.float32),
    )(a0, a1, g2, dinv, b2)


def kernel(x, edge_index, W1, b1, W2, b2):
    n = x.shape[0]
    f1 = W1.shape[1]
    f2 = W2.shape[1]

    src = edge_index[0].astype(jnp.int32)
    dst = edge_index[1].astype(jnp.int32)
    pad = E_PAD - src.shape[0]
    pad_src = jnp.arange(pad, dtype=jnp.int32) % n
    pad_dst = TRASH + jnp.arange(pad, dtype=jnp.int32) % (N_PAD - N_NODES)
    src2d = jnp.concatenate([src, pad_src]).reshape(R_EDGE, CH)
    dst2d = jnp.concatenate([dst, pad_dst]).reshape(R_EDGE, CH)

    W2p = jnp.pad(W2, ((0, 0), (0, f1 - f2)))

    ones_hbm = jnp.ones((CH,), jnp.float32)
    zdeg = jnp.zeros((N_PAD,), jnp.float32)
    z1 = jnp.zeros((SLICE, f1), jnp.float32)

    deg_parts = _sc_degree(dst2d, ones_hbm, zdeg).reshape(NC, N_PAD)
    p0 = deg_parts[0, :n, None]
    p1 = deg_parts[1, :n, None]

    g1, dinv = _tc_layer1(x, W1, p0, p1)

    acc1 = _sc_aggregate(g1, src2d, dst2d, z1, f1)
    g2 = _tc_layer2(acc1[0, :n], acc1[1, :n], g1, dinv,
                    b1.reshape(1, f1), W2p)

    acc2 = _sc_aggregate(g2, src2d, dst2d, z1, f1)
    out = _tc_final(acc2[0, :n], acc2[1, :n], g2, dinv, b2.reshape(1, f2), f2)
    return out


# speedup vs baseline: 30.0642x; 1.4086x over previous
import functools

import jax
import jax.numpy as jnp
from jax import lax
from jax.experimental import pallas as pl
from jax.experimental.pallas import tpu as pltpu
from jax.experimental.pallas import tpu_sc as plsc

N_NODES = 10000
N_EDGES = 320000

NC = 2
NS = 16
NW = NC * NS
CH = 128

RPW = ((N_EDGES + NW * CH - 1) // (NW * CH) + 7) // 8 * 8
R_EDGE = RPW * NW
SEGS = 2
SEG_ROWS = RPW // SEGS
E_PAD = R_EDGE * CH

N_PAD = 10240
SLICE = N_PAD // NS
TRASH = N_NODES
F = 128

_mesh = plsc.VectorSubcoreMesh(core_axis_name="c", subcore_axis_name="s",
                               num_cores=NC, num_subcores=NS)


def _sc_degree(dst2d, ones_hbm, zeros_hbm):
    @functools.partial(
        pl.kernel,
        out_type=jax.ShapeDtypeStruct((NC * N_PAD,), jnp.float32),
        mesh=_mesh,
        scratch_types=[
            pltpu.VMEM((RPW, CH), jnp.int32),
            pltpu.VMEM((CH,), jnp.float32),
            pltpu.VMEM_SHARED((N_PAD,), jnp.float32),
        ],
    )
    def k(dst_hbm, ones_h, zeros_h, out_hbm, idx_v, ones_v, acc_s):
        c = lax.axis_index("c")
        s = lax.axis_index("s")
        w = s * NC + c
        pltpu.sync_copy(zeros_h.at[pl.ds(s * SLICE, SLICE)],
                        acc_s.at[pl.ds(s * SLICE, SLICE)])
        pltpu.sync_copy(ones_h, ones_v)
        pltpu.sync_copy(dst_hbm.at[pl.ds(w * RPW, RPW)], idx_v)
        plsc.subcore_barrier()

        @pl.loop(0, RPW)
        def _(j):
            pltpu.sync_copy(ones_v, acc_s.at[idx_v.at[j]], add=True)

        plsc.subcore_barrier()
        pltpu.sync_copy(acc_s.at[pl.ds(s * SLICE, SLICE)],
                        out_hbm.at[pl.ds(c * N_PAD + s * SLICE, SLICE)])

    return k(dst2d, ones_hbm, zeros_hbm)


def _sc_aggregate(g, src2d, dst2d, zeros_hbm):
    @functools.partial(
        pl.kernel,
        out_type=jax.ShapeDtypeStruct((NC, N_PAD, F), jnp.float32),
        mesh=_mesh,
        scratch_types=[
            pltpu.VMEM((SEG_ROWS, CH), jnp.int32),
            pltpu.VMEM((SEG_ROWS, CH), jnp.int32),
            pltpu.VMEM((CH, F), jnp.float32),
            pltpu.VMEM((CH, F), jnp.float32),
            pltpu.VMEM_SHARED((N_PAD, F), jnp.float32),
            pltpu.SemaphoreType.DMA,
            pltpu.SemaphoreType.DMA,
        ],
    )
    def k(g_hbm, src_hbm, dst_hbm, zeros_h, out_hbm,
          isrc_v, idst_v, rows0_v, rows1_v, acc_s, sem0, sem1):
        c = lax.axis_index("c")
        s = lax.axis_index("s")
        w = s * NC + c
        pltpu.sync_copy(zeros_h, acc_s.at[pl.ds(s * SLICE, SLICE)])
        plsc.subcore_barrier()

        for seg in range(SEGS):
            base = w * RPW + seg * SEG_ROWS
            pltpu.sync_copy(src_hbm.at[pl.ds(base, SEG_ROWS)], isrc_v)
            pltpu.sync_copy(dst_hbm.at[pl.ds(base, SEG_ROWS)], idst_v)

            pltpu.async_copy(g_hbm.at[isrc_v.at[0]], rows0_v, sem0)
            pltpu.async_copy(g_hbm.at[isrc_v.at[1]], rows1_v, sem1)

            @pl.loop(0, SEG_ROWS, step=2)
            def _(j):
                pltpu.make_async_copy(g_hbm.at[isrc_v.at[0]],
                                      rows0_v, sem0).wait()
                pltpu.sync_copy(rows0_v, acc_s.at[idst_v.at[j]], add=True)

                @pl.when(j + 2 < SEG_ROWS)
                def _():
                    pltpu.async_copy(g_hbm.at[isrc_v.at[j + 2]], rows0_v, sem0)

                pltpu.make_async_copy(g_hbm.at[isrc_v.at[0]],
                                      rows1_v, sem1).wait()
                pltpu.sync_copy(rows1_v, acc_s.at[idst_v.at[j + 1]], add=True)

                @pl.when(j + 3 < SEG_ROWS)
                def _():
                    pltpu.async_copy(g_hbm.at[isrc_v.at[j + 3]], rows1_v, sem1)

        plsc.subcore_barrier()
        pltpu.sync_copy(acc_s.at[pl.ds(s * SLICE, SLICE)],
                        out_hbm.at[c, pl.ds(s * SLICE, SLICE)])

    return k(g, src2d, dst2d, zeros_hbm)


_BLK = 2000


def _tc_layer1(x, W1, p0, p1):
    def body(x_ref, w_ref, p0_ref, p1_ref, g_ref, dinv_ref):
        deg = p0_ref[...] + p1_ref[...] + 1.0
        dinv = lax.rsqrt(deg)
        h = jnp.dot(x_ref[...], w_ref[...],
                    preferred_element_type=jnp.float32,
                    precision=lax.Precision.HIGHEST)
        g_ref[...] = h * dinv
        dinv_ref[...] = dinv

    n, f = x.shape
    fo = W1.shape[1]
    return pl.pallas_call(
        body,
        grid=(n // _BLK,),
        in_specs=[
            pl.BlockSpec((_BLK, f), lambda i: (i, 0)),
            pl.BlockSpec((f, fo), lambda i: (0, 0)),
            pl.BlockSpec((_BLK, 1), lambda i: (i, 0)),
            pl.BlockSpec((_BLK, 1), lambda i: (i, 0)),
        ],
        out_specs=[
            pl.BlockSpec((_BLK, fo), lambda i: (i, 0)),
            pl.BlockSpec((_BLK, 1), lambda i: (i, 0)),
        ],
        out_shape=[
            jax.ShapeDtypeStruct((n, fo), jnp.float32),
            jax.ShapeDtypeStruct((n, 1), jnp.float32),
        ],
    )(x, W1, p0, p1)


def _tc_layer2(a0, a1, g1, dinv, b1, W2p):
    def body(a0_ref, a1_ref, g1_ref, dinv_ref, b1_ref, w_ref, out_ref):
        dinv = dinv_ref[...]
        pre = (a0_ref[...] + a1_ref[...] + g1_ref[...]) * dinv + b1_ref[...]
        h = jnp.maximum(pre, 0.0)
        out_ref[...] = jnp.dot(h, w_ref[...],
                               preferred_element_type=jnp.float32,
                               precision=lax.Precision.HIGHEST) * dinv

    n, f = g1.shape
    fo = W2p.shape[1]
    return pl.pallas_call(
        body,
        grid=(n // _BLK,),
        in_specs=[
            pl.BlockSpec((_BLK, f), lambda i: (i, 0)),
            pl.BlockSpec((_BLK, f), lambda i: (i, 0)),
            pl.BlockSpec((_BLK, f), lambda i: (i, 0)),
            pl.BlockSpec((_BLK, 1), lambda i: (i, 0)),
            pl.BlockSpec((1, f), lambda i: (0, 0)),
            pl.BlockSpec((f, fo), lambda i: (0, 0)),
        ],
        out_specs=pl.BlockSpec((_BLK, fo), lambda i: (i, 0)),
        out_shape=jax.ShapeDtypeStruct((n, fo), jnp.float32),
    )(a0, a1, g1, dinv, b1, W2p)


def _tc_final(a0, a1, g2, dinv, b2, fo):
    def body(a0_ref, a1_ref, g2_ref, dinv_ref, b2_ref, out_ref):
        s = (a0_ref[...] + a1_ref[...] + g2_ref[...])[:, :fo]
        out_ref[...] = s * dinv_ref[...] + b2_ref[...]

    n, f = g2.shape
    return pl.pallas_call(
        body,
        grid=(n // _BLK,),
        in_specs=[
            pl.BlockSpec((_BLK, f), lambda i: (i, 0)),
            pl.BlockSpec((_BLK, f), lambda i: (i, 0)),
            pl.BlockSpec((_BLK, f), lambda i: (i, 0)),
            pl.BlockSpec((_BLK, 1), lambda i: (i, 0)),
            pl.BlockSpec((1, fo), lambda i: (0, 0)),
        ],
        out_specs=pl.BlockSpec((_BLK, fo), lambda i: (i, 0)),
        out_shape=jax.ShapeDtypeStruct((n, fo), jnp.float32),
    )(a0, a1, g2, dinv, b2)


def kernel(x, edge_index, W1, b1, W2, b2):
    n = x.shape[0]
    f1 = W1.shape[1]
    f2 = W2.shape[1]

    src = edge_index[0].astype(jnp.int32)
    dst = edge_index[1].astype(jnp.int32)
    pad = E_PAD - src.shape[0]
    pad_src = jnp.arange(pad, dtype=jnp.int32) % n
    pad_dst = TRASH + jnp.arange(pad, dtype=jnp.int32) % (N_PAD - N_NODES)
    src2d = jnp.concatenate([src, pad_src]).reshape(R_EDGE, CH)
    dst2d = jnp.concatenate([dst, pad_dst]).reshape(R_EDGE, CH)

    W2p = jnp.pad(W2, ((0, 0), (0, f1 - f2)))

    ones_hbm = jnp.ones((CH,), jnp.float32)
    zdeg = jnp.zeros((N_PAD,), jnp.float32)
    zrow = jnp.zeros((SLICE, F), jnp.float32)

    deg_parts = _sc_degree(dst2d, ones_hbm, zdeg).reshape(NC, N_PAD)
    p0 = deg_parts[0, :n, None]
    p1 = deg_parts[1, :n, None]

    g1, dinv = _tc_layer1(x, W1, p0, p1)

    acc1 = _sc_aggregate(g1, src2d, dst2d, zrow)
    g2 = _tc_layer2(acc1[0, :n], acc1[1, :n], g1, dinv,
                    b1.reshape(1, f1), W2p)

    acc2 = _sc_aggregate(g2, src2d, dst2d, zrow)
    out = _tc_final(acc2[0, :n], acc2[1, :n], g2, dinv, b2.reshape(1, f2), f2)
    return out


# speedup vs baseline: 32.1725x; 1.0701x over previous
import functools

import jax
import jax.numpy as jnp
from jax import lax
from jax.experimental import pallas as pl
from jax.experimental.pallas import tpu as pltpu
from jax.experimental.pallas import tpu_sc as plsc

N_NODES = 10000
N_EDGES = 320000

NC = 2
NS = 16
NW = NC * NS
CH = 128

RPW = ((N_EDGES + NW * CH - 1) // (NW * CH) + 7) // 8 * 8
R_EDGE = RPW * NW
SEGS = 2
SEG_ROWS = RPW // SEGS
E_PAD = R_EDGE * CH

N_PAD = 10240
SLICE = N_PAD // NS
TRASH = N_NODES
F = 128

_mesh = plsc.VectorSubcoreMesh(core_axis_name="c", subcore_axis_name="s",
                               num_cores=NC, num_subcores=NS)


def _sc_degree(dst2d, ones_hbm, zeros_hbm):
    @functools.partial(
        pl.kernel,
        out_type=jax.ShapeDtypeStruct((NC * N_PAD,), jnp.float32),
        mesh=_mesh,
        scratch_types=[
            pltpu.VMEM((RPW, CH), jnp.int32),
            pltpu.VMEM((CH,), jnp.float32),
            pltpu.VMEM_SHARED((N_PAD,), jnp.float32),
        ],
    )
    def k(dst_hbm, ones_h, zeros_h, out_hbm, idx_v, ones_v, acc_s):
        c = lax.axis_index("c")
        s = lax.axis_index("s")
        w = s * NC + c
        pltpu.sync_copy(zeros_h.at[pl.ds(s * SLICE, SLICE)],
                        acc_s.at[pl.ds(s * SLICE, SLICE)])
        pltpu.sync_copy(ones_h, ones_v)
        pltpu.sync_copy(dst_hbm.at[pl.ds(w * RPW, RPW)], idx_v)
        plsc.subcore_barrier()

        @pl.loop(0, RPW)
        def _(j):
            pltpu.sync_copy(ones_v, acc_s.at[idx_v.at[j]], add=True)

        plsc.subcore_barrier()
        pltpu.sync_copy(acc_s.at[pl.ds(s * SLICE, SLICE)],
                        out_hbm.at[pl.ds(c * N_PAD + s * SLICE, SLICE)])

    return k(dst2d, ones_hbm, zeros_hbm)


def _sc_aggregate(g, src2d, dst2d):
    @functools.partial(
        pl.kernel,
        out_type=jax.ShapeDtypeStruct((NC, N_PAD, F), jnp.float32),
        mesh=_mesh,
        scratch_types=[
            pltpu.VMEM((SEG_ROWS, CH), jnp.int32),
            pltpu.VMEM((SEG_ROWS, CH), jnp.int32),
            pltpu.VMEM((CH, F), jnp.float32),
            pltpu.VMEM((CH, F), jnp.float32),
            pltpu.VMEM_SHARED((N_PAD, F), jnp.float32),
            pltpu.SemaphoreType.DMA,
            pltpu.SemaphoreType.DMA,
        ],
    )
    def k(g_hbm, src_hbm, dst_hbm, out_hbm,
          isrc_v, idst_v, rows0_v, rows1_v, acc_s, sem0, sem1):
        c = lax.axis_index("c")
        s = lax.axis_index("s")
        w = s * NC + c

        zv = jnp.zeros((16,), jnp.float32)

        @pl.loop(0, CH)
        def _(r):
            @pl.loop(0, F, step=16)
            def _(cc):
                rows0_v[r, pl.ds(cc, 16)] = zv

        @pl.loop(0, SLICE, step=CH)
        def _(r0):
            pltpu.sync_copy(rows0_v, acc_s.at[pl.ds(s * SLICE + r0, CH)])

        plsc.subcore_barrier()

        for seg in range(SEGS):
            base = w * RPW + seg * SEG_ROWS
            pltpu.sync_copy(src_hbm.at[pl.ds(base, SEG_ROWS)], isrc_v)
            pltpu.sync_copy(dst_hbm.at[pl.ds(base, SEG_ROWS)], idst_v)

            pltpu.async_copy(g_hbm.at[isrc_v.at[0]], rows0_v, sem0)
            pltpu.async_copy(g_hbm.at[isrc_v.at[1]], rows1_v, sem1)

            @pl.loop(0, SEG_ROWS, step=2)
            def _(j):
                pltpu.make_async_copy(g_hbm.at[isrc_v.at[0]],
                                      rows0_v, sem0).wait()
                pltpu.sync_copy(rows0_v, acc_s.at[idst_v.at[j]], add=True)

                @pl.when(j + 2 < SEG_ROWS)
                def _():
                    pltpu.async_copy(g_hbm.at[isrc_v.at[j + 2]], rows0_v, sem0)

                pltpu.make_async_copy(g_hbm.at[isrc_v.at[0]],
                                      rows1_v, sem1).wait()
                pltpu.sync_copy(rows1_v, acc_s.at[idst_v.at[j + 1]], add=True)

                @pl.when(j + 3 < SEG_ROWS)
                def _():
                    pltpu.async_copy(g_hbm.at[isrc_v.at[j + 3]], rows1_v, sem1)

        plsc.subcore_barrier()
        pltpu.sync_copy(acc_s.at[pl.ds(s * SLICE, SLICE)],
                        out_hbm.at[c, pl.ds(s * SLICE, SLICE)])

    return k(g, src2d, dst2d)


_BLK = 2000


def _tc_matmul(x, W1):
    def body(x_ref, w_ref, h_ref):
        h_ref[...] = jnp.dot(x_ref[...], w_ref[...],
                             preferred_element_type=jnp.float32,
                             precision=lax.Precision.HIGHEST)

    n, f = x.shape
    fo = W1.shape[1]
    return pl.pallas_call(
        body,
        grid=(n // _BLK,),
        in_specs=[
            pl.BlockSpec((_BLK, f), lambda i: (i, 0)),
            pl.BlockSpec((f, fo), lambda i: (0, 0)),
        ],
        out_specs=pl.BlockSpec((_BLK, fo), lambda i: (i, 0)),
        out_shape=jax.ShapeDtypeStruct((n, fo), jnp.float32),
    )(x, W1)


def _tc_scale(h, p0, p1):
    def body(h_ref, p0_ref, p1_ref, g_ref, dinv_ref):
        dinv = lax.rsqrt(p0_ref[...] + p1_ref[...] + 1.0)
        g_ref[...] = h_ref[...] * dinv
        dinv_ref[...] = dinv

    n, f = h.shape
    return pl.pallas_call(
        body,
        grid=(n // _BLK,),
        in_specs=[
            pl.BlockSpec((_BLK, f), lambda i: (i, 0)),
            pl.BlockSpec((_BLK, 1), lambda i: (i, 0)),
            pl.BlockSpec((_BLK, 1), lambda i: (i, 0)),
        ],
        out_specs=[
            pl.BlockSpec((_BLK, f), lambda i: (i, 0)),
            pl.BlockSpec((_BLK, 1), lambda i: (i, 0)),
        ],
        out_shape=[
            jax.ShapeDtypeStruct((n, f), jnp.float32),
            jax.ShapeDtypeStruct((n, 1), jnp.float32),
        ],
    )(h, p0, p1)


def _tc_layer2(acc1, g1, dinv, b1, W2p):
    def body(a0_ref, a1_ref, g1_ref, dinv_ref, b1_ref, w_ref, out_ref):
        dinv = dinv_ref[...]
        pre = (a0_ref[0] + a1_ref[0] + g1_ref[...]) * dinv + b1_ref[...]
        h = jnp.maximum(pre, 0.0)
        out_ref[...] = jnp.dot(h, w_ref[...],
                               preferred_element_type=jnp.float32,
                               precision=lax.Precision.HIGHEST) * dinv

    n, f = g1.shape
    fo = W2p.shape[1]
    part = lambda cidx: pl.BlockSpec((1, _BLK, f), lambda i, c=cidx: (c, i, 0))
    return pl.pallas_call(
        body,
        grid=(n // _BLK,),
        in_specs=[
            part(0),
            part(1),
            pl.BlockSpec((_BLK, f), lambda i: (i, 0)),
            pl.BlockSpec((_BLK, 1), lambda i: (i, 0)),
            pl.BlockSpec((1, f), lambda i: (0, 0)),
            pl.BlockSpec((f, fo), lambda i: (0, 0)),
        ],
        out_specs=pl.BlockSpec((_BLK, fo), lambda i: (i, 0)),
        out_shape=jax.ShapeDtypeStruct((n, fo), jnp.float32),
    )(acc1, acc1, g1, dinv, b1, W2p)


def _tc_final(acc2, g2, dinv, b2, fo):
    def body(a0_ref, a1_ref, g2_ref, dinv_ref, b2_ref, out_ref):
        s = (a0_ref[0] + a1_ref[0] + g2_ref[...])[:, :fo]
        out_ref[...] = s * dinv_ref[...] + b2_ref[...]

    n, f = g2.shape
    part = lambda cidx: pl.BlockSpec((1, _BLK, f), lambda i, c=cidx: (c, i, 0))
    return pl.pallas_call(
        body,
        grid=(n // _BLK,),
        in_specs=[
            part(0),
            part(1),
            pl.BlockSpec((_BLK, f), lambda i: (i, 0)),
            pl.BlockSpec((_BLK, 1), lambda i: (i, 0)),
            pl.BlockSpec((1, fo), lambda i: (0, 0)),
        ],
        out_specs=pl.BlockSpec((_BLK, fo), lambda i: (i, 0)),
        out_shape=jax.ShapeDtypeStruct((n, fo), jnp.float32),
    )(acc2, acc2, g2, dinv, b2)


def kernel(x, edge_index, W1, b1, W2, b2):
    n = x.shape[0]
    f1 = W1.shape[1]
    f2 = W2.shape[1]

    src = edge_index[0].astype(jnp.int32)
    dst = edge_index[1].astype(jnp.int32)
    pad = E_PAD - src.shape[0]
    pad_src = jnp.arange(pad, dtype=jnp.int32) % n
    pad_dst = TRASH + jnp.arange(pad, dtype=jnp.int32) % (N_PAD - N_NODES)
    src2d = jnp.concatenate([src, pad_src]).reshape(R_EDGE, CH)
    dst2d = jnp.concatenate([dst, pad_dst]).reshape(R_EDGE, CH)

    W2p = jnp.pad(W2, ((0, 0), (0, f1 - f2)))

    ones_hbm = jnp.ones((CH,), jnp.float32)
    zdeg = jnp.zeros((N_PAD,), jnp.float32)

    deg_parts = _sc_degree(dst2d, ones_hbm, zdeg).reshape(NC, N_PAD)
    p0 = deg_parts[0, :n, None]
    p1 = deg_parts[1, :n, None]

    h1 = _tc_matmul(x, W1)
    g1, dinv = _tc_scale(h1, p0, p1)

    acc1 = _sc_aggregate(g1, src2d, dst2d)
    g2 = _tc_layer2(acc1, g1, dinv, b1.reshape(1, f1), W2p)

    acc2 = _sc_aggregate(g2, src2d, dst2d)
    out = _tc_final(acc2, g2, dinv, b2.reshape(1, f2), f2)
    return out


# speedup vs baseline: 35.2112x; 1.0945x over previous
import functools

import jax
import jax.numpy as jnp
from jax import lax
from jax.experimental import pallas as pl
from jax.experimental.pallas import tpu as pltpu
from jax.experimental.pallas import tpu_sc as plsc

N_NODES = 10000
N_EDGES = 320000

NC = 2
NS = 16
NW = NC * NS
CH = 128

RPW = ((N_EDGES + NW * CH - 1) // (NW * CH) + 7) // 8 * 8
R_EDGE = RPW * NW
SEGS = 2
SEG_ROWS = RPW // SEGS
E_PAD = R_EDGE * CH

N_PAD = 10240
SLICE = N_PAD // NS
TRASH = N_NODES
F = 128

_mesh = plsc.VectorSubcoreMesh(core_axis_name="c", subcore_axis_name="s",
                               num_cores=NC, num_subcores=NS)


def _sc_degree(dst2d, ones_hbm, zeros_hbm):
    @functools.partial(
        pl.kernel,
        out_type=jax.ShapeDtypeStruct((NC * N_PAD,), jnp.float32),
        mesh=_mesh,
        scratch_types=[
            pltpu.VMEM((RPW, CH), jnp.int32),
            pltpu.VMEM((CH,), jnp.float32),
            pltpu.VMEM_SHARED((N_PAD,), jnp.float32),
        ],
    )
    def k(dst_hbm, ones_h, zeros_h, out_hbm, idx_v, ones_v, acc_s):
        c = lax.axis_index("c")
        s = lax.axis_index("s")
        w = s * NC + c
        pltpu.sync_copy(zeros_h.at[pl.ds(s * SLICE, SLICE)],
                        acc_s.at[pl.ds(s * SLICE, SLICE)])
        pltpu.sync_copy(ones_h, ones_v)
        pltpu.sync_copy(dst_hbm.at[pl.ds(w * RPW, RPW)], idx_v)
        plsc.subcore_barrier()

        @pl.loop(0, RPW)
        def _(j):
            pltpu.sync_copy(ones_v, acc_s.at[idx_v.at[j]], add=True)

        plsc.subcore_barrier()
        pltpu.sync_copy(acc_s.at[pl.ds(s * SLICE, SLICE)],
                        out_hbm.at[pl.ds(c * N_PAD + s * SLICE, SLICE)])

    return k(dst2d, ones_hbm, zeros_hbm)


def _sc_aggregate(g, src2d, dst2d, fw=F, tc_tiling=True):
    cp = pltpu.CompilerParams(use_tc_tiling_on_sc=tc_tiling)

    @functools.partial(
        pl.kernel,
        out_type=jax.ShapeDtypeStruct((NC, N_PAD, fw), jnp.float32),
        mesh=_mesh,
        compiler_params=cp,
        scratch_types=[
            pltpu.VMEM((SEG_ROWS, CH), jnp.int32),
            pltpu.VMEM((SEG_ROWS, CH), jnp.int32),
            pltpu.VMEM((CH, fw), jnp.float32),
            pltpu.VMEM((CH, fw), jnp.float32),
            pltpu.VMEM_SHARED((N_PAD, fw), jnp.float32),
            pltpu.SemaphoreType.DMA,
            pltpu.SemaphoreType.DMA,
        ],
    )
    def k(g_hbm, src_hbm, dst_hbm, out_hbm,
          isrc_v, idst_v, rows0_v, rows1_v, acc_s, sem0, sem1):
        c = lax.axis_index("c")
        s = lax.axis_index("s")
        w = s * NC + c

        zv = jnp.zeros((16,), jnp.float32)

        @pl.loop(0, CH)
        def _(r):
            @pl.loop(0, fw, step=16)
            def _(cc):
                rows0_v[r, pl.ds(cc, 16)] = zv

        @pl.loop(0, SLICE, step=CH)
        def _(r0):
            pltpu.sync_copy(rows0_v, acc_s.at[pl.ds(s * SLICE + r0, CH)])

        plsc.subcore_barrier()

        for seg in range(SEGS):
            base = w * RPW + seg * SEG_ROWS
            pltpu.sync_copy(src_hbm.at[pl.ds(base, SEG_ROWS)], isrc_v)
            pltpu.sync_copy(dst_hbm.at[pl.ds(base, SEG_ROWS)], idst_v)

            pltpu.async_copy(g_hbm.at[isrc_v.at[0]], rows0_v, sem0)
            pltpu.async_copy(g_hbm.at[isrc_v.at[1]], rows1_v, sem1)

            @pl.loop(0, SEG_ROWS, step=2)
            def _(j):
                pltpu.make_async_copy(g_hbm.at[isrc_v.at[0]],
                                      rows0_v, sem0).wait()
                pltpu.sync_copy(rows0_v, acc_s.at[idst_v.at[j]], add=True)

                @pl.when(j + 2 < SEG_ROWS)
                def _():
                    pltpu.async_copy(g_hbm.at[isrc_v.at[j + 2]], rows0_v, sem0)

                pltpu.make_async_copy(g_hbm.at[isrc_v.at[0]],
                                      rows1_v, sem1).wait()
                pltpu.sync_copy(rows1_v, acc_s.at[idst_v.at[j + 1]], add=True)

                @pl.when(j + 3 < SEG_ROWS)
                def _():
                    pltpu.async_copy(g_hbm.at[isrc_v.at[j + 3]], rows1_v, sem1)

        plsc.subcore_barrier()
        pltpu.sync_copy(acc_s.at[pl.ds(s * SLICE, SLICE)],
                        out_hbm.at[c, pl.ds(s * SLICE, SLICE)])

    return k(g, src2d, dst2d)


_BLK = 2000


def _tc_matmul(x, W1):
    def body(x_ref, w_ref, h_ref):
        h_ref[...] = jnp.dot(x_ref[...], w_ref[...],
                             preferred_element_type=jnp.float32,
                             precision=lax.Precision.HIGHEST)

    n, f = x.shape
    fo = W1.shape[1]
    return pl.pallas_call(
        body,
        grid=(n // _BLK,),
        in_specs=[
            pl.BlockSpec((_BLK, f), lambda i: (i, 0)),
            pl.BlockSpec((f, fo), lambda i: (0, 0)),
        ],
        out_specs=pl.BlockSpec((_BLK, fo), lambda i: (i, 0)),
        out_shape=jax.ShapeDtypeStruct((n, fo), jnp.float32),
    )(x, W1)


def _tc_scale(h, p0, p1):
    def body(h_ref, p0_ref, p1_ref, g_ref, dinv_ref):
        dinv = lax.rsqrt(p0_ref[...] + p1_ref[...] + 1.0)
        g_ref[...] = h_ref[...] * dinv
        dinv_ref[...] = dinv

    n, f = h.shape
    return pl.pallas_call(
        body,
        grid=(n // _BLK,),
        in_specs=[
            pl.BlockSpec((_BLK, f), lambda i: (i, 0)),
            pl.BlockSpec((_BLK, 1), lambda i: (i, 0)),
            pl.BlockSpec((_BLK, 1), lambda i: (i, 0)),
        ],
        out_specs=[
            pl.BlockSpec((_BLK, f), lambda i: (i, 0)),
            pl.BlockSpec((_BLK, 1), lambda i: (i, 0)),
        ],
        out_shape=[
            jax.ShapeDtypeStruct((n, f), jnp.float32),
            jax.ShapeDtypeStruct((n, 1), jnp.float32),
        ],
    )(h, p0, p1)


def _tc_layer2(acc1, g1, dinv, b1, W2p):
    def body(a0_ref, a1_ref, g1_ref, dinv_ref, b1_ref, w_ref, out_ref):
        dinv = dinv_ref[...]
        pre = (a0_ref[0] + a1_ref[0] + g1_ref[...]) * dinv + b1_ref[...]
        h = jnp.maximum(pre, 0.0)
        out_ref[...] = jnp.dot(h, w_ref[...],
                               preferred_element_type=jnp.float32,
                               precision=lax.Precision.HIGHEST) * dinv

    n, f = g1.shape
    fo = W2p.shape[1]
    part = lambda cidx: pl.BlockSpec((1, _BLK, f), lambda i, c=cidx: (c, i, 0))
    return pl.pallas_call(
        body,
        grid=(n // _BLK,),
        in_specs=[
            part(0),
            part(1),
            pl.BlockSpec((_BLK, f), lambda i: (i, 0)),
            pl.BlockSpec((_BLK, 1), lambda i: (i, 0)),
            pl.BlockSpec((1, f), lambda i: (0, 0)),
            pl.BlockSpec((f, fo), lambda i: (0, 0)),
        ],
        out_specs=pl.BlockSpec((_BLK, fo), lambda i: (i, 0)),
        out_shape=jax.ShapeDtypeStruct((n, fo), jnp.float32),
    )(acc1, acc1, g1, dinv, b1, W2p)


def _tc_final(acc2, g2, dinv, b2):
    def body(a0_ref, a1_ref, g2_ref, dinv_ref, b2_ref, out_ref):
        s = a0_ref[0] + a1_ref[0] + g2_ref[...]
        out_ref[...] = s * dinv_ref[...] + b2_ref[...]

    n, f = g2.shape
    fo = f
    part = lambda cidx: pl.BlockSpec((1, _BLK, f), lambda i, c=cidx: (c, i, 0))
    return pl.pallas_call(
        body,
        grid=(n // _BLK,),
        in_specs=[
            part(0),
            part(1),
            pl.BlockSpec((_BLK, f), lambda i: (i, 0)),
            pl.BlockSpec((_BLK, 1), lambda i: (i, 0)),
            pl.BlockSpec((1, fo), lambda i: (0, 0)),
        ],
        out_specs=pl.BlockSpec((_BLK, fo), lambda i: (i, 0)),
        out_shape=jax.ShapeDtypeStruct((n, fo), jnp.float32),
    )(acc2, acc2, g2, dinv, b2)


def kernel(x, edge_index, W1, b1, W2, b2):
    n = x.shape[0]
    f1 = W1.shape[1]
    f2 = W2.shape[1]

    src = edge_index[0].astype(jnp.int32)
    dst = edge_index[1].astype(jnp.int32)
    pad = E_PAD - src.shape[0]
    pad_src = jnp.arange(pad, dtype=jnp.int32) % n
    pad_dst = TRASH + jnp.arange(pad, dtype=jnp.int32) % (N_PAD - N_NODES)
    src2d = jnp.concatenate([src, pad_src]).reshape(R_EDGE, CH)
    dst2d = jnp.concatenate([dst, pad_dst]).reshape(R_EDGE, CH)

    ones_hbm = jnp.ones((CH,), jnp.float32)
    zdeg = jnp.zeros((N_PAD,), jnp.float32)

    deg_parts = _sc_degree(dst2d, ones_hbm, zdeg).reshape(NC, N_PAD)
    p0 = deg_parts[0, :n, None]
    p1 = deg_parts[1, :n, None]

    h1 = _tc_matmul(x, W1)
    g1, dinv = _tc_scale(h1, p0, p1)

    acc1 = _sc_aggregate(g1, src2d, dst2d)
    g2 = _tc_layer2(acc1, g1, dinv, b1.reshape(1, f1), W2)

    acc2 = _sc_aggregate(g2, src2d, dst2d, fw=f2, tc_tiling=False)
    out = _tc_final(acc2, g2, dinv, b2.reshape(1, f2))
    return out


# speedup vs baseline: 36.1591x; 1.0269x over previous
import functools

import jax
import jax.numpy as jnp
from jax import lax
from jax.experimental import pallas as pl
from jax.experimental.pallas import tpu as pltpu
from jax.experimental.pallas import tpu_sc as plsc

N_NODES = 10000
N_EDGES = 320000

NC = 2
NS = 16
NW = NC * NS
CH = 128

RPW = ((N_EDGES + NW * CH - 1) // (NW * CH) + 7) // 8 * 8
R_EDGE = RPW * NW
SEGS = 2
SEG_ROWS = RPW // SEGS
E_PAD = R_EDGE * CH

N_PAD = 10240
SLICE = N_PAD // NS
TRASH = N_NODES
F = 128

_mesh = plsc.VectorSubcoreMesh(core_axis_name="c", subcore_axis_name="s",
                               num_cores=NC, num_subcores=NS)


def _sc_degree(dst2d, ones_hbm, zeros_hbm):
    @functools.partial(
        pl.kernel,
        out_type=jax.ShapeDtypeStruct((NC * N_PAD,), jnp.float32),
        mesh=_mesh,
        scratch_types=[
            pltpu.VMEM((RPW, CH), jnp.int32),
            pltpu.VMEM((CH,), jnp.float32),
            pltpu.VMEM_SHARED((N_PAD,), jnp.float32),
            pltpu.SemaphoreType.DMA,
        ],
    )
    def k(dst_hbm, ones_h, zeros_h, out_hbm, idx_v, ones_v, acc_s, sem):
        c = lax.axis_index("c")
        s = lax.axis_index("s")
        w = s * NC + c
        pltpu.sync_copy(zeros_h.at[pl.ds(s * SLICE, SLICE)],
                        acc_s.at[pl.ds(s * SLICE, SLICE)])
        pltpu.sync_copy(ones_h, ones_v)
        pltpu.sync_copy(dst_hbm.at[pl.ds(w * RPW, RPW)], idx_v)
        plsc.subcore_barrier()

        GRP = 16

        @pl.loop(0, RPW, step=GRP)
        def _(j0):
            @pl.loop(0, GRP)
            def _(i):
                pltpu.async_copy(ones_v, acc_s.at[idx_v.at[j0 + i]], sem,
                                 add=True)

            @pl.loop(0, GRP)
            def _(i):
                pltpu.make_async_copy(ones_v, acc_s.at[idx_v.at[0]],
                                      sem).wait()

        plsc.subcore_barrier()
        pltpu.sync_copy(acc_s.at[pl.ds(s * SLICE, SLICE)],
                        out_hbm.at[pl.ds(c * N_PAD + s * SLICE, SLICE)])

    return k(dst2d, ones_hbm, zeros_hbm)


def _sc_aggregate(g, src2d, dst2d, fw=F, tc_tiling=True):
    cp = pltpu.CompilerParams(use_tc_tiling_on_sc=tc_tiling)

    @functools.partial(
        pl.kernel,
        out_type=jax.ShapeDtypeStruct((NC, N_PAD, fw), jnp.float32),
        mesh=_mesh,
        compiler_params=cp,
        scratch_types=[
            pltpu.VMEM((SEG_ROWS, CH), jnp.int32),
            pltpu.VMEM((SEG_ROWS, CH), jnp.int32),
            pltpu.VMEM((CH, fw), jnp.float32),
            pltpu.VMEM((CH, fw), jnp.float32),
            pltpu.VMEM_SHARED((N_PAD, fw), jnp.float32),
            pltpu.SemaphoreType.DMA,
            pltpu.SemaphoreType.DMA,
        ],
    )
    def k(g_hbm, src_hbm, dst_hbm, out_hbm,
          isrc_v, idst_v, rows0_v, rows1_v, acc_s, sem0, sem1):
        c = lax.axis_index("c")
        s = lax.axis_index("s")
        w = s * NC + c

        zv = jnp.zeros((16,), jnp.float32)

        @pl.loop(0, CH)
        def _(r):
            @pl.loop(0, fw, step=16)
            def _(cc):
                rows0_v[r, pl.ds(cc, 16)] = zv

        @pl.loop(0, SLICE, step=CH)
        def _(r0):
            pltpu.sync_copy(rows0_v, acc_s.at[pl.ds(s * SLICE + r0, CH)])

        plsc.subcore_barrier()

        for seg in range(SEGS):
            base = w * RPW + seg * SEG_ROWS
            pltpu.sync_copy(src_hbm.at[pl.ds(base, SEG_ROWS)], isrc_v)
            pltpu.sync_copy(dst_hbm.at[pl.ds(base, SEG_ROWS)], idst_v)

            pltpu.async_copy(g_hbm.at[isrc_v.at[0]], rows0_v, sem0)
            pltpu.async_copy(g_hbm.at[isrc_v.at[1]], rows1_v, sem1)

            @pl.loop(0, SEG_ROWS, step=2)
            def _(j):
                pltpu.make_async_copy(g_hbm.at[isrc_v.at[0]],
                                      rows0_v, sem0).wait()
                pltpu.sync_copy(rows0_v, acc_s.at[idst_v.at[j]], add=True)

                @pl.when(j + 2 < SEG_ROWS)
                def _():
                    pltpu.async_copy(g_hbm.at[isrc_v.at[j + 2]], rows0_v, sem0)

                pltpu.make_async_copy(g_hbm.at[isrc_v.at[0]],
                                      rows1_v, sem1).wait()
                pltpu.sync_copy(rows1_v, acc_s.at[idst_v.at[j + 1]], add=True)

                @pl.when(j + 3 < SEG_ROWS)
                def _():
                    pltpu.async_copy(g_hbm.at[isrc_v.at[j + 3]], rows1_v, sem1)

        plsc.subcore_barrier()
        pltpu.sync_copy(acc_s.at[pl.ds(s * SLICE, SLICE)],
                        out_hbm.at[c, pl.ds(s * SLICE, SLICE)])

    return k(g, src2d, dst2d)


_BLK = 2000

R_REAL = N_EDGES // CH


def _tc_edges(ei3, n):
    def body(s_ref, d_ref, src_ref, dst_ref):
        src_ref[:R_REAL] = s_ref[0]
        dst_ref[:R_REAL] = d_ref[0]
        flat = (lax.broadcasted_iota(jnp.int32, (R_EDGE - R_REAL, CH), 0) * CH
                + lax.broadcasted_iota(jnp.int32, (R_EDGE - R_REAL, CH), 1))
        src_ref[R_REAL:] = flat % n
        dst_ref[R_REAL:] = TRASH + flat % (N_PAD - N_NODES)

    part = lambda cidx: pl.BlockSpec((1, R_REAL, CH),
                                     lambda i, c=cidx: (c, 0, 0))
    return pl.pallas_call(
        body,
        grid=(1,),
        in_specs=[part(0), part(1)],
        out_specs=[
            pl.BlockSpec((R_EDGE, CH), lambda i: (0, 0)),
            pl.BlockSpec((R_EDGE, CH), lambda i: (0, 0)),
        ],
        out_shape=[
            jax.ShapeDtypeStruct((R_EDGE, CH), jnp.int32),
            jax.ShapeDtypeStruct((R_EDGE, CH), jnp.int32),
        ],
    )(ei3, ei3)


def _tc_matmul(x, W1):
    def body(x_ref, w_ref, h_ref):
        h_ref[...] = jnp.dot(x_ref[...], w_ref[...],
                             preferred_element_type=jnp.float32,
                             precision=lax.Precision.HIGHEST)

    n, f = x.shape
    fo = W1.shape[1]
    return pl.pallas_call(
        body,
        grid=(n // _BLK,),
        in_specs=[
            pl.BlockSpec((_BLK, f), lambda i: (i, 0)),
            pl.BlockSpec((f, fo), lambda i: (0, 0)),
        ],
        out_specs=pl.BlockSpec((_BLK, fo), lambda i: (i, 0)),
        out_shape=jax.ShapeDtypeStruct((n, fo), jnp.float32),
    )(x, W1)


def _tc_scale(h, deg3):
    def body(h_ref, p0_ref, p1_ref, g_ref, dinv_ref):
        dinv = lax.rsqrt(p0_ref[0] + p1_ref[0] + 1.0)
        g_ref[...] = h_ref[...] * dinv
        dinv_ref[...] = dinv

    n, f = h.shape
    part = lambda cidx: pl.BlockSpec((1, _BLK, 1), lambda i, c=cidx: (c, i, 0))
    return pl.pallas_call(
        body,
        grid=(n // _BLK,),
        in_specs=[
            pl.BlockSpec((_BLK, f), lambda i: (i, 0)),
            part(0),
            part(1),
        ],
        out_specs=[
            pl.BlockSpec((_BLK, f), lambda i: (i, 0)),
            pl.BlockSpec((_BLK, 1), lambda i: (i, 0)),
        ],
        out_shape=[
            jax.ShapeDtypeStruct((n, f), jnp.float32),
            jax.ShapeDtypeStruct((n, 1), jnp.float32),
        ],
    )(h, deg3, deg3)


def _tc_layer2(acc1, g1, dinv, b1, W2p):
    def body(a0_ref, a1_ref, g1_ref, dinv_ref, b1_ref, w_ref, out_ref):
        dinv = dinv_ref[...]
        pre = (a0_ref[0] + a1_ref[0] + g1_ref[...]) * dinv + b1_ref[...]
        h = jnp.maximum(pre, 0.0)
        out_ref[...] = jnp.dot(h, w_ref[...],
                               preferred_element_type=jnp.float32,
                               precision=lax.Precision.HIGHEST) * dinv

    n, f = g1.shape
    fo = W2p.shape[1]
    part = lambda cidx: pl.BlockSpec((1, _BLK, f), lambda i, c=cidx: (c, i, 0))
    return pl.pallas_call(
        body,
        grid=(n // _BLK,),
        in_specs=[
            part(0),
            part(1),
            pl.BlockSpec((_BLK, f), lambda i: (i, 0)),
            pl.BlockSpec((_BLK, 1), lambda i: (i, 0)),
            pl.BlockSpec((1, f), lambda i: (0, 0)),
            pl.BlockSpec((f, fo), lambda i: (0, 0)),
        ],
        out_specs=pl.BlockSpec((_BLK, fo), lambda i: (i, 0)),
        out_shape=jax.ShapeDtypeStruct((n, fo), jnp.float32),
    )(acc1, acc1, g1, dinv, b1, W2p)


def _tc_final(acc2, g2, dinv, b2):
    def body(a0_ref, a1_ref, g2_ref, dinv_ref, b2_ref, out_ref):
        s = a0_ref[0] + a1_ref[0] + g2_ref[...]
        out_ref[...] = s * dinv_ref[...] + b2_ref[...]

    n, f = g2.shape
    fo = f
    part = lambda cidx: pl.BlockSpec((1, _BLK, f), lambda i, c=cidx: (c, i, 0))
    return pl.pallas_call(
        body,
        grid=(n // _BLK,),
        in_specs=[
            part(0),
            part(1),
            pl.BlockSpec((_BLK, f), lambda i: (i, 0)),
            pl.BlockSpec((_BLK, 1), lambda i: (i, 0)),
            pl.BlockSpec((1, fo), lambda i: (0, 0)),
        ],
        out_specs=pl.BlockSpec((_BLK, fo), lambda i: (i, 0)),
        out_shape=jax.ShapeDtypeStruct((n, fo), jnp.float32),
    )(acc2, acc2, g2, dinv, b2)


def kernel(x, edge_index, W1, b1, W2, b2):
    n = x.shape[0]
    f1 = W1.shape[1]
    f2 = W2.shape[1]

    ei3 = edge_index.astype(jnp.int32).reshape(2, R_REAL, CH)
    src2d, dst2d = _tc_edges(ei3, n)

    ones_hbm = jnp.ones((CH,), jnp.float32)
    zdeg = jnp.zeros((N_PAD,), jnp.float32)

    deg3 = _sc_degree(dst2d, ones_hbm, zdeg).reshape(NC, N_PAD, 1)

    h1 = _tc_matmul(x, W1)
    g1, dinv = _tc_scale(h1, deg3)

    acc1 = _sc_aggregate(g1, src2d, dst2d)
    g2 = _tc_layer2(acc1, g1, dinv, b1.reshape(1, f1), W2)

    acc2 = _sc_aggregate(g2, src2d, dst2d, fw=f2, tc_tiling=False)
    out = _tc_final(acc2, g2, dinv, b2.reshape(1, f2))
    return out
